# baseline (speedup 1.0000x reference)
import functools

import jax
import jax.numpy as jnp
from jax import lax
from jax.experimental import pallas as pl
from jax.experimental.pallas import tpu as pltpu
from jax.experimental.pallas import tpu_sc as plsc

B = 1024
CTX_DIM = 512
PREFIX_LEN = 5
SUFFIX_LEN = 40
N_CLS = 4
N_ATTR = 7
N_ATTR_SLOTS = 4 * N_ATTR
SEQ = PREFIX_LEN + N_CLS + N_ATTR_SLOTS + SUFFIX_LEN
A0 = PREFIX_LEN + N_CLS
S0 = SEQ - SUFFIX_LEN

N_WORKERS = 32
B_PER_W = B // N_WORKERS
NPLANE = 2
LANES = 16


def _sc_assemble(label, attr_sel, pre_pad, suf_pad, small, cls_ctx):
    mesh = plsc.VectorSubcoreMesh(core_axis_name="c", subcore_axis_name="s")

    @functools.partial(
        pl.kernel,
        out_type=jax.ShapeDtypeStruct((B, SEQ, CTX_DIM), jnp.float32),
        mesh=mesh,
        scratch_types=[
            pltpu.VMEM((B_PER_W, 1), jnp.int32),
            pltpu.VMEM((B_PER_W, N_ATTR), jnp.int32),
            pltpu.VMEM((NPLANE, SEQ, CTX_DIM), jnp.float32),
            pltpu.VMEM((1, N_CLS, CTX_DIM), jnp.float32),
            pltpu.VMEM((N_ATTR, 4, CTX_DIM), jnp.float32),
            pltpu.SemaphoreType.DMA,
            pltpu.SemaphoreType.DMA,
        ],
    )
    def body(label_hbm, sel_hbm, pre_hbm, suf_hbm, small_hbm, cls_hbm,
             out_hbm, lab_v, sel_v, planes, cstg, astg, isem, wsem):
        wid = lax.axis_index("s") * 2 + lax.axis_index("c")
        base = wid * B_PER_W

        def issue_fills(i):
            pltpu.async_copy(cls_hbm.at[lab_v.at[i]], cstg, isem)
            pltpu.async_copy(small_hbm.at[sel_v.at[i]], astg, isem)

        def wait_fills():
            pltpu.make_async_copy(cls_hbm.at[pl.ds(0, 1)], cstg, isem).wait()
            pltpu.make_async_copy(small_hbm.at[pl.ds(0, N_ATTR)], astg,
                                  isem).wait()

        def drain_write():
            pltpu.make_async_copy(planes.at[0], out_hbm.at[0], wsem).wait()

        pltpu.sync_copy(label_hbm.at[pl.ds(base, B_PER_W)], lab_v)
        pltpu.sync_copy(sel_hbm.at[pl.ds(base, B_PER_W)], sel_v)
        for s in range(NPLANE):
            pltpu.sync_copy(pre_hbm, planes.at[s, pl.ds(0, 8)])
            for k in range(5):
                pltpu.sync_copy(suf_hbm.at[pl.ds(8 * k, 8)],
                                planes.at[s, pl.ds(32 + 8 * k, 8)])
            pltpu.sync_copy(suf_hbm.at[pl.ds(40, 5)],
                            planes.at[s, pl.ds(72, 5)])
        issue_fills(0)

        @pl.loop(0, B_PER_W)
        def _(i):
            s = lax.rem(i, NPLANE)

            wait_fills()
            @pl.when(i >= NPLANE)
            def _():
                drain_write()

            @pl.loop(0, N_CLS)
            def _(r):
                for l in range(CTX_DIM // LANES):
                    planes[s, PREFIX_LEN + r, pl.ds(l * LANES, LANES)] = (
                        cstg[0, r, pl.ds(l * LANES, LANES)])

            @pl.loop(0, N_ATTR_SLOTS)
            def _(r):
                a = lax.div(r, 4)
                j = lax.rem(r, 4)
                for l in range(CTX_DIM // LANES):
                    planes[s, A0 + r, pl.ds(l * LANES, LANES)] = (
                        astg[a, j, pl.ds(l * LANES, LANES)])

            @pl.when(i < B_PER_W - 1)
            def _():
                issue_fills(i + 1)

            pltpu.async_copy(planes.at[s], out_hbm.at[base + i], wsem)

        for _ in range(NPLANE):
            drain_write()

    return body(label, attr_sel, pre_pad, suf_pad, small, cls_ctx)


def kernel(get_train, label, gender_idx, ucc_idx, ucs_idx, lcc_idx, lcs_idx,
           hat_idx, backpack_idx, token_prefix, token_suffix, gender_ctx,
           ucc_ctx, ucs_ctx, lcc_ctx, lcs_ctx, hat_ctx, backpack_ctx,
           cls_ctx):
    del get_train
    tables = (gender_ctx, ucc_ctx, ucs_ctx, lcc_ctx, lcs_ctx, hat_ctx,
              backpack_ctx)
    idxs = (gender_idx, ucc_idx, ucs_idx, lcc_idx, lcs_idx, hat_idx,
            backpack_idx)
    small = jnp.concatenate(tables, axis=0)

    sel_parts = []
    entry_base = 0
    for t, ix in zip(tables, idxs):
        sel_parts.append(entry_base + ix.astype(jnp.int32)[:, None])
        entry_base += t.shape[0]
    attr_sel = jnp.concatenate(sel_parts, axis=1)

    prefix = token_prefix.reshape(PREFIX_LEN, CTX_DIM)
    suffix = token_suffix.reshape(SUFFIX_LEN, CTX_DIM)
    zeros3 = jnp.zeros((3, CTX_DIM), jnp.float32)
    zeros5 = jnp.zeros((5, CTX_DIM), jnp.float32)
    pre_pad = jnp.concatenate([prefix, zeros3], axis=0)
    suf_pad = jnp.concatenate([zeros5, suffix, zeros3], axis=0)

    return _sc_assemble(label.astype(jnp.int32).reshape(B, 1), attr_sel,
                        pre_pad, suf_pad,
                        small, cls_ctx)


# speedup vs baseline: 1.1268x; 1.1268x over previous
import functools

import jax
import jax.numpy as jnp
from jax import lax
from jax.experimental import pallas as pl
from jax.experimental.pallas import tpu as pltpu
from jax.experimental.pallas import tpu_sc as plsc

B = 1024
CTX_DIM = 512
PREFIX_LEN = 5
SUFFIX_LEN = 40
N_CLS = 4
N_ATTR = 7
N_ATTR_SLOTS = 4 * N_ATTR
SEQ = PREFIX_LEN + N_CLS + N_ATTR_SLOTS + SUFFIX_LEN
A0 = PREFIX_LEN + N_CLS
S0 = SEQ - SUFFIX_LEN

N_WORKERS = 32
B_PER_W = B // N_WORKERS
NPLANE = 2
LANES = 16


def _sc_assemble(label, attr_sel, pre_pad, suf_pad, small, cls_ctx):
    mesh = plsc.VectorSubcoreMesh(core_axis_name="c", subcore_axis_name="s")

    @functools.partial(
        pl.kernel,
        out_type=jax.ShapeDtypeStruct((B, SEQ, CTX_DIM), jnp.float32),
        mesh=mesh,
        scratch_types=[
            pltpu.VMEM((B_PER_W, 1), jnp.int32),
            pltpu.VMEM((B_PER_W, N_ATTR), jnp.int32),
            pltpu.VMEM((NPLANE, SEQ, CTX_DIM), jnp.float32),
            pltpu.VMEM((2, 1, N_CLS, CTX_DIM), jnp.float32),
            pltpu.VMEM((2, N_ATTR, 4, CTX_DIM), jnp.float32),
            pltpu.SemaphoreType.DMA((2,)),
            pltpu.SemaphoreType.DMA,
        ],
    )
    def body(label_hbm, sel_hbm, pre_hbm, suf_hbm, small_hbm, cls_hbm,
             out_hbm, lab_v, sel_v, planes, cstg, astg, isem, wsem):
        wid = lax.axis_index("s") * 2 + lax.axis_index("c")
        base = wid * B_PER_W

        def issue_fills(i, k):
            pltpu.async_copy(cls_hbm.at[lab_v.at[i]], cstg.at[k], isem.at[k])
            pltpu.async_copy(small_hbm.at[sel_v.at[i]], astg.at[k],
                             isem.at[k])

        def wait_fills(k):
            pltpu.make_async_copy(cls_hbm.at[pl.ds(0, 1)], cstg.at[0],
                                  isem.at[k]).wait()
            pltpu.make_async_copy(small_hbm.at[pl.ds(0, N_ATTR)], astg.at[0],
                                  isem.at[k]).wait()

        def drain_write():
            pltpu.make_async_copy(planes.at[0], out_hbm.at[0], wsem).wait()

        pltpu.sync_copy(label_hbm.at[pl.ds(base, B_PER_W)], lab_v)
        pltpu.sync_copy(sel_hbm.at[pl.ds(base, B_PER_W)], sel_v)
        for s in range(NPLANE):
            pltpu.sync_copy(pre_hbm, planes.at[s, pl.ds(0, 8)])
            for k in range(5):
                pltpu.sync_copy(suf_hbm.at[pl.ds(8 * k, 8)],
                                planes.at[s, pl.ds(32 + 8 * k, 8)])
            pltpu.sync_copy(suf_hbm.at[pl.ds(40, 5)],
                            planes.at[s, pl.ds(72, 5)])
        issue_fills(0, 0)
        issue_fills(1, 1)

        @pl.loop(0, B_PER_W)
        def _(i):
            s = lax.rem(i, NPLANE)

            wait_fills(s)
            @pl.when(i >= NPLANE)
            def _():
                drain_write()

            @pl.loop(0, N_CLS)
            def _(r):
                for l in range(CTX_DIM // LANES):
                    planes[s, PREFIX_LEN + r, pl.ds(l * LANES, LANES)] = (
                        cstg[s, 0, r, pl.ds(l * LANES, LANES)])

            @pl.loop(0, N_ATTR_SLOTS)
            def _(r):
                a = lax.div(r, 4)
                j = lax.rem(r, 4)
                for l in range(CTX_DIM // LANES):
                    planes[s, A0 + r, pl.ds(l * LANES, LANES)] = (
                        astg[s, a, j, pl.ds(l * LANES, LANES)])

            @pl.when(i < B_PER_W - 2)
            def _():
                issue_fills(i + 2, s)

            pltpu.async_copy(planes.at[s], out_hbm.at[base + i], wsem)

        for _ in range(NPLANE):
            drain_write()

    return body(label, attr_sel, pre_pad, suf_pad, small, cls_ctx)


def kernel(get_train, label, gender_idx, ucc_idx, ucs_idx, lcc_idx, lcs_idx,
           hat_idx, backpack_idx, token_prefix, token_suffix, gender_ctx,
           ucc_ctx, ucs_ctx, lcc_ctx, lcs_ctx, hat_ctx, backpack_ctx,
           cls_ctx):
    del get_train
    tables = (gender_ctx, ucc_ctx, ucs_ctx, lcc_ctx, lcs_ctx, hat_ctx,
              backpack_ctx)
    idxs = (gender_idx, ucc_idx, ucs_idx, lcc_idx, lcs_idx, hat_idx,
            backpack_idx)
    small = jnp.concatenate(tables, axis=0)

    sel_parts = []
    entry_base = 0
    for t, ix in zip(tables, idxs):
        sel_parts.append(entry_base + ix.astype(jnp.int32)[:, None])
        entry_base += t.shape[0]
    attr_sel = jnp.concatenate(sel_parts, axis=1)

    prefix = token_prefix.reshape(PREFIX_LEN, CTX_DIM)
    suffix = token_suffix.reshape(SUFFIX_LEN, CTX_DIM)
    zeros3 = jnp.zeros((3, CTX_DIM), jnp.float32)
    zeros5 = jnp.zeros((5, CTX_DIM), jnp.float32)
    pre_pad = jnp.concatenate([prefix, zeros3], axis=0)
    suf_pad = jnp.concatenate([zeros5, suffix, zeros3], axis=0)

    return _sc_assemble(label.astype(jnp.int32).reshape(B, 1), attr_sel,
                        pre_pad, suf_pad,
                        small, cls_ctx)


# speedup vs baseline: 1.5551x; 1.3801x over previous
import functools

import jax
import jax.numpy as jnp
from jax import lax
from jax.experimental import pallas as pl
from jax.experimental.pallas import tpu as pltpu
from jax.experimental.pallas import tpu_sc as plsc

B = 1024
CTX_DIM = 512
PREFIX_LEN = 5
SUFFIX_LEN = 40
N_CLS = 4
N_ATTR_SLOTS = 28
SMALL_ROWS = 176
SEQ = PREFIX_LEN + N_CLS + N_ATTR_SLOTS + SUFFIX_LEN

NB = 16
N_WORKERS = 32
B_PER_W = B // N_WORKERS


def _sc_cls_gather(label, cls_ctx):
    mesh = plsc.VectorSubcoreMesh(core_axis_name="c", subcore_axis_name="s")

    @functools.partial(
        pl.kernel,
        out_type=jax.ShapeDtypeStruct((B, N_CLS, CTX_DIM), jnp.float32),
        mesh=mesh,
        scratch_types=[
            pltpu.VMEM((B_PER_W,), jnp.int32),
            pltpu.VMEM((B_PER_W, N_CLS, CTX_DIM), jnp.float32),
            pltpu.SemaphoreType.DMA,
        ],
    )
    def gather_kernel(label_hbm, cls_hbm, out_hbm, idx_v, rows_v, sem):
        wid = lax.axis_index("s") * 2 + lax.axis_index("c")
        base = wid * B_PER_W
        pltpu.sync_copy(label_hbm.at[pl.ds(base, B_PER_W)], idx_v)
        pltpu.async_copy(cls_hbm.at[idx_v], rows_v, sem).wait()
        pltpu.sync_copy(rows_v, out_hbm.at[pl.ds(base, B_PER_W)])

    return gather_kernel(label, cls_ctx)


NRING = 6
GRID = B // NB


def _tc_assemble_body(cls_ref, pre_ref, suf_ref, hi_ref, lo_ref, sel_ref,
                      out_ref, stage, sem):
    i = pl.program_id(0)
    slot = lax.rem(i, NRING)

    @pl.when(i >= NRING)
    def _():
        pltpu.make_async_copy(stage.at[slot], out_ref.at[pl.ds(0, NB)],
                              sem.at[slot]).wait()

    st = stage.at[slot]
    st[:, 0:PREFIX_LEN, :] = jnp.broadcast_to(
        pre_ref[...][None], (NB, PREFIX_LEN, CTX_DIM))
    st[:, PREFIX_LEN:PREFIX_LEN + N_CLS, :] = cls_ref[...]
    st[:, SEQ - SUFFIX_LEN:SEQ, :] = jnp.broadcast_to(
        suf_ref[...][None], (NB, SUFFIX_LEN, CTX_DIM))
    sel = sel_ref[...]
    onehot = (sel == lax.broadcasted_iota(jnp.int32, (1, SMALL_ROWS), 1))
    onehot = onehot.astype(jnp.bfloat16)
    res = (jnp.dot(onehot, hi_ref[...], preferred_element_type=jnp.float32)
           + jnp.dot(onehot, lo_ref[...], preferred_element_type=jnp.float32))
    a0 = PREFIX_LEN + N_CLS
    for b in range(NB):
        st[b, a0:a0 + N_ATTR_SLOTS, :] = (
            res[b * N_ATTR_SLOTS:(b + 1) * N_ATTR_SLOTS, :])

    pltpu.make_async_copy(stage.at[slot], out_ref.at[pl.ds(i * NB, NB)],
                          sem.at[slot]).start()

    @pl.when(i == GRID - 1)
    def _():
        for r in range(NRING):
            pltpu.make_async_copy(stage.at[r], out_ref.at[pl.ds(0, NB)],
                                  sem.at[r]).wait()


def _tc_assemble(cls_part, prefix, suffix, small_hi, small_lo, attr_sel):
    return pl.pallas_call(
        _tc_assemble_body,
        grid=(GRID,),
        in_specs=[
            pl.BlockSpec((NB, N_CLS, CTX_DIM), lambda i: (i, 0, 0)),
            pl.BlockSpec((PREFIX_LEN, CTX_DIM), lambda i: (0, 0)),
            pl.BlockSpec((SUFFIX_LEN, CTX_DIM), lambda i: (0, 0)),
            pl.BlockSpec((SMALL_ROWS, CTX_DIM), lambda i: (0, 0)),
            pl.BlockSpec((SMALL_ROWS, CTX_DIM), lambda i: (0, 0)),
            pl.BlockSpec((NB * N_ATTR_SLOTS, 1), lambda i: (i, 0)),
        ],
        out_specs=pl.BlockSpec(memory_space=pl.ANY),
        out_shape=jax.ShapeDtypeStruct((B, SEQ, CTX_DIM), jnp.float32),
        scratch_shapes=[
            pltpu.VMEM((NRING, NB, SEQ, CTX_DIM), jnp.float32),
            pltpu.SemaphoreType.DMA((NRING,)),
        ],
    )(cls_part, prefix, suffix, small_hi, small_lo, attr_sel)


def kernel(get_train, label, gender_idx, ucc_idx, ucs_idx, lcc_idx, lcs_idx,
           hat_idx, backpack_idx, token_prefix, token_suffix, gender_ctx,
           ucc_ctx, ucs_ctx, lcc_ctx, lcs_ctx, hat_ctx, backpack_ctx,
           cls_ctx):
    del get_train
    tables = (gender_ctx, ucc_ctx, ucs_ctx, lcc_ctx, lcs_ctx, hat_ctx,
              backpack_ctx)
    idxs = (gender_idx, ucc_idx, ucs_idx, lcc_idx, lcs_idx, hat_idx,
            backpack_idx)
    small = jnp.concatenate([t.reshape(-1, CTX_DIM) for t in tables], axis=0)
    small_hi = small.astype(jnp.bfloat16)
    small_lo = (small - small_hi.astype(jnp.float32)).astype(jnp.bfloat16)

    sel_parts = []
    base = 0
    j = jnp.arange(4, dtype=jnp.int32)[None, :]
    for t, ix in zip(tables, idxs):
        sel_parts.append(base + 4 * ix.astype(jnp.int32)[:, None] + j)
        base += t.shape[0] * 4
    attr_sel = jnp.concatenate(sel_parts, axis=1).reshape(
        B * N_ATTR_SLOTS, 1)

    prefix = token_prefix.reshape(PREFIX_LEN, CTX_DIM)
    suffix = token_suffix.reshape(SUFFIX_LEN, CTX_DIM)

    cls_part = _sc_cls_gather(label.astype(jnp.int32), cls_ctx)

    return _tc_assemble(cls_part, prefix, suffix, small_hi, small_lo,
                        attr_sel)


# speedup vs baseline: 1.6555x; 1.0646x over previous
import functools

import jax
import jax.numpy as jnp
from jax import lax
from jax.experimental import pallas as pl
from jax.experimental.pallas import tpu as pltpu
from jax.experimental.pallas import tpu_sc as plsc

B = 1024
CTX_DIM = 512
PREFIX_LEN = 5
SUFFIX_LEN = 40
N_CLS = 4
N_ATTR_SLOTS = 28
SMALL_ROWS = 176
SEQ = PREFIX_LEN + N_CLS + N_ATTR_SLOTS + SUFFIX_LEN

NB = 32
N_WORKERS = 32
B_PER_W = B // N_WORKERS


def _sc_cls_gather(label, cls_ctx):
    mesh = plsc.VectorSubcoreMesh(core_axis_name="c", subcore_axis_name="s")

    @functools.partial(
        pl.kernel,
        out_type=jax.ShapeDtypeStruct((B, N_CLS, CTX_DIM), jnp.float32),
        mesh=mesh,
        scratch_types=[
            pltpu.VMEM((B_PER_W,), jnp.int32),
            pltpu.VMEM((B_PER_W, N_CLS, CTX_DIM), jnp.float32),
            pltpu.SemaphoreType.DMA,
        ],
    )
    def gather_kernel(label_hbm, cls_hbm, out_hbm, idx_v, rows_v, sem):
        wid = lax.axis_index("s") * 2 + lax.axis_index("c")
        base = wid * B_PER_W
        pltpu.sync_copy(label_hbm.at[pl.ds(base, B_PER_W)], idx_v)
        pltpu.async_copy(cls_hbm.at[idx_v], rows_v, sem).wait()
        pltpu.sync_copy(rows_v, out_hbm.at[pl.ds(base, B_PER_W)])

    return gather_kernel(label, cls_ctx)


NRING = 6
GRID = B // NB


def _tc_assemble_body(cls_ref, pre_ref, suf_ref, hi_ref, lo_ref, sel_ref,
                      out_ref, stage, sem):
    i = pl.program_id(0)
    slot = lax.rem(i, NRING)

    @pl.when(i >= NRING)
    def _():
        pltpu.make_async_copy(stage.at[slot], out_ref.at[pl.ds(0, NB)],
                              sem.at[slot]).wait()

    st = stage.at[slot]
    st[:, 0:PREFIX_LEN, :] = jnp.broadcast_to(
        pre_ref[...][None], (NB, PREFIX_LEN, CTX_DIM))
    st[:, PREFIX_LEN:PREFIX_LEN + N_CLS, :] = cls_ref[...]
    st[:, SEQ - SUFFIX_LEN:SEQ, :] = jnp.broadcast_to(
        suf_ref[...][None], (NB, SUFFIX_LEN, CTX_DIM))
    sel = sel_ref[...]
    onehot = (sel == lax.broadcasted_iota(jnp.int32, (1, SMALL_ROWS), 1))
    onehot = onehot.astype(jnp.bfloat16)
    res = (jnp.dot(onehot, hi_ref[...], preferred_element_type=jnp.float32)
           + jnp.dot(onehot, lo_ref[...], preferred_element_type=jnp.float32))
    a0 = PREFIX_LEN + N_CLS
    for b in range(NB):
        st[b, a0:a0 + N_ATTR_SLOTS, :] = (
            res[b * N_ATTR_SLOTS:(b + 1) * N_ATTR_SLOTS, :])

    pltpu.make_async_copy(stage.at[slot], out_ref.at[pl.ds(i * NB, NB)],
                          sem.at[slot]).start()

    @pl.when(i == GRID - 1)
    def _():
        for r in range(NRING):
            pltpu.make_async_copy(stage.at[r], out_ref.at[pl.ds(0, NB)],
                                  sem.at[r]).wait()


def _tc_assemble(cls_part, prefix, suffix, small_hi, small_lo, attr_sel):
    return pl.pallas_call(
        _tc_assemble_body,
        grid=(GRID,),
        in_specs=[
            pl.BlockSpec((NB, N_CLS, CTX_DIM), lambda i: (i, 0, 0)),
            pl.BlockSpec((PREFIX_LEN, CTX_DIM), lambda i: (0, 0)),
            pl.BlockSpec((SUFFIX_LEN, CTX_DIM), lambda i: (0, 0)),
            pl.BlockSpec((SMALL_ROWS, CTX_DIM), lambda i: (0, 0)),
            pl.BlockSpec((SMALL_ROWS, CTX_DIM), lambda i: (0, 0)),
            pl.BlockSpec((NB * N_ATTR_SLOTS, 1), lambda i: (i, 0)),
        ],
        out_specs=pl.BlockSpec(memory_space=pl.ANY),
        out_shape=jax.ShapeDtypeStruct((B, SEQ, CTX_DIM), jnp.float32),
        scratch_shapes=[
            pltpu.VMEM((NRING, NB, SEQ, CTX_DIM), jnp.float32),
            pltpu.SemaphoreType.DMA((NRING,)),
        ],
    )(cls_part, prefix, suffix, small_hi, small_lo, attr_sel)


def kernel(get_train, label, gender_idx, ucc_idx, ucs_idx, lcc_idx, lcs_idx,
           hat_idx, backpack_idx, token_prefix, token_suffix, gender_ctx,
           ucc_ctx, ucs_ctx, lcc_ctx, lcs_ctx, hat_ctx, backpack_ctx,
           cls_ctx):
    del get_train
    tables = (gender_ctx, ucc_ctx, ucs_ctx, lcc_ctx, lcs_ctx, hat_ctx,
              backpack_ctx)
    idxs = (gender_idx, ucc_idx, ucs_idx, lcc_idx, lcs_idx, hat_idx,
            backpack_idx)
    small = jnp.concatenate([t.reshape(-1, CTX_DIM) for t in tables], axis=0)
    small_hi = small.astype(jnp.bfloat16)
    small_lo = (small - small_hi.astype(jnp.float32)).astype(jnp.bfloat16)

    sel_parts = []
    base = 0
    j = jnp.arange(4, dtype=jnp.int32)[None, :]
    for t, ix in zip(tables, idxs):
        sel_parts.append(base + 4 * ix.astype(jnp.int32)[:, None] + j)
        base += t.shape[0] * 4
    attr_sel = jnp.concatenate(sel_parts, axis=1).reshape(
        B * N_ATTR_SLOTS, 1)

    prefix = token_prefix.reshape(PREFIX_LEN, CTX_DIM)
    suffix = token_suffix.reshape(SUFFIX_LEN, CTX_DIM)

    cls_part = _sc_cls_gather(label.astype(jnp.int32), cls_ctx)

    return _tc_assemble(cls_part, prefix, suffix, small_hi, small_lo,
                        attr_sel)


# speedup vs baseline: 1.6664x; 1.0066x over previous
import functools

import jax
import jax.numpy as jnp
from jax import lax
from jax.experimental import pallas as pl
from jax.experimental.pallas import tpu as pltpu
from jax.experimental.pallas import tpu_sc as plsc

B = 1024
CTX_DIM = 512
PREFIX_LEN = 5
SUFFIX_LEN = 40
N_CLS = 4
N_ATTR_SLOTS = 28
SMALL_ROWS = 176
SEQ = PREFIX_LEN + N_CLS + N_ATTR_SLOTS + SUFFIX_LEN

NB = 64
N_WORKERS = 32
B_PER_W = B // N_WORKERS


def _sc_cls_gather(label, cls_ctx):
    mesh = plsc.VectorSubcoreMesh(core_axis_name="c", subcore_axis_name="s")

    @functools.partial(
        pl.kernel,
        out_type=jax.ShapeDtypeStruct((B, N_CLS, CTX_DIM), jnp.float32),
        mesh=mesh,
        scratch_types=[
            pltpu.VMEM((B_PER_W,), jnp.int32),
            pltpu.VMEM((B_PER_W, N_CLS, CTX_DIM), jnp.float32),
            pltpu.SemaphoreType.DMA,
        ],
    )
    def gather_kernel(label_hbm, cls_hbm, out_hbm, idx_v, rows_v, sem):
        wid = lax.axis_index("s") * 2 + lax.axis_index("c")
        base = wid * B_PER_W
        pltpu.sync_copy(label_hbm.at[pl.ds(base, B_PER_W)], idx_v)
        pltpu.async_copy(cls_hbm.at[idx_v], rows_v, sem).wait()
        pltpu.sync_copy(rows_v, out_hbm.at[pl.ds(base, B_PER_W)])

    return gather_kernel(label, cls_ctx)


NRING = 4
GRID = B // NB


def _tc_assemble_body(cls_ref, pre_ref, suf_ref, hi_ref, lo_ref, sel_ref,
                      out_ref, stage, sem):
    i = pl.program_id(0)
    slot = lax.rem(i, NRING)

    @pl.when(i >= NRING)
    def _():
        pltpu.make_async_copy(stage.at[slot], out_ref.at[pl.ds(0, NB)],
                              sem.at[slot]).wait()

    st = stage.at[slot]
    st[:, 0:PREFIX_LEN, :] = jnp.broadcast_to(
        pre_ref[...][None], (NB, PREFIX_LEN, CTX_DIM))
    st[:, PREFIX_LEN:PREFIX_LEN + N_CLS, :] = cls_ref[...]
    st[:, SEQ - SUFFIX_LEN:SEQ, :] = jnp.broadcast_to(
        suf_ref[...][None], (NB, SUFFIX_LEN, CTX_DIM))
    sel = sel_ref[...]
    onehot = (sel == lax.broadcasted_iota(jnp.int32, (1, SMALL_ROWS), 1))
    onehot = onehot.astype(jnp.bfloat16)
    res = (jnp.dot(onehot, hi_ref[...], preferred_element_type=jnp.float32)
           + jnp.dot(onehot, lo_ref[...], preferred_element_type=jnp.float32))
    a0 = PREFIX_LEN + N_CLS
    for b in range(NB):
        st[b, a0:a0 + N_ATTR_SLOTS, :] = (
            res[b * N_ATTR_SLOTS:(b + 1) * N_ATTR_SLOTS, :])

    pltpu.make_async_copy(stage.at[slot], out_ref.at[pl.ds(i * NB, NB)],
                          sem.at[slot]).start()

    @pl.when(i == GRID - 1)
    def _():
        for r in range(NRING):
            pltpu.make_async_copy(stage.at[r], out_ref.at[pl.ds(0, NB)],
                                  sem.at[r]).wait()


def _tc_assemble(cls_part, prefix, suffix, small_hi, small_lo, attr_sel):
    return pl.pallas_call(
        _tc_assemble_body,
        grid=(GRID,),
        in_specs=[
            pl.BlockSpec((NB, N_CLS, CTX_DIM), lambda i: (i, 0, 0)),
            pl.BlockSpec((PREFIX_LEN, CTX_DIM), lambda i: (0, 0)),
            pl.BlockSpec((SUFFIX_LEN, CTX_DIM), lambda i: (0, 0)),
            pl.BlockSpec((SMALL_ROWS, CTX_DIM), lambda i: (0, 0)),
            pl.BlockSpec((SMALL_ROWS, CTX_DIM), lambda i: (0, 0)),
            pl.BlockSpec((NB * N_ATTR_SLOTS, 1), lambda i: (i, 0)),
        ],
        out_specs=pl.BlockSpec(memory_space=pl.ANY),
        out_shape=jax.ShapeDtypeStruct((B, SEQ, CTX_DIM), jnp.float32),
        scratch_shapes=[
            pltpu.VMEM((NRING, NB, SEQ, CTX_DIM), jnp.float32),
            pltpu.SemaphoreType.DMA((NRING,)),
        ],
    )(cls_part, prefix, suffix, small_hi, small_lo, attr_sel)


def kernel(get_train, label, gender_idx, ucc_idx, ucs_idx, lcc_idx, lcs_idx,
           hat_idx, backpack_idx, token_prefix, token_suffix, gender_ctx,
           ucc_ctx, ucs_ctx, lcc_ctx, lcs_ctx, hat_ctx, backpack_ctx,
           cls_ctx):
    del get_train
    tables = (gender_ctx, ucc_ctx, ucs_ctx, lcc_ctx, lcs_ctx, hat_ctx,
              backpack_ctx)
    idxs = (gender_idx, ucc_idx, ucs_idx, lcc_idx, lcs_idx, hat_idx,
            backpack_idx)
    small = jnp.concatenate([t.reshape(-1, CTX_DIM) for t in tables], axis=0)
    small_hi = small.astype(jnp.bfloat16)
    small_lo = (small - small_hi.astype(jnp.float32)).astype(jnp.bfloat16)

    sel_parts = []
    base = 0
    j = jnp.arange(4, dtype=jnp.int32)[None, :]
    for t, ix in zip(tables, idxs):
        sel_parts.append(base + 4 * ix.astype(jnp.int32)[:, None] + j)
        base += t.shape[0] * 4
    attr_sel = jnp.concatenate(sel_parts, axis=1).reshape(
        B * N_ATTR_SLOTS, 1)

    prefix = token_prefix.reshape(PREFIX_LEN, CTX_DIM)
    suffix = token_suffix.reshape(SUFFIX_LEN, CTX_DIM)

    cls_part = _sc_cls_gather(label.astype(jnp.int32), cls_ctx)

    return _tc_assemble(cls_part, prefix, suffix, small_hi, small_lo,
                        attr_sel)


# speedup vs baseline: 1.6681x; 1.0010x over previous
import functools

import jax
import jax.numpy as jnp
from jax import lax
from jax.experimental import pallas as pl
from jax.experimental.pallas import tpu as pltpu
from jax.experimental.pallas import tpu_sc as plsc

B = 1024
CTX_DIM = 512
PREFIX_LEN = 5
SUFFIX_LEN = 40
N_CLS = 4
N_ATTR_SLOTS = 28
SMALL_ROWS = 176
SEQ = PREFIX_LEN + N_CLS + N_ATTR_SLOTS + SUFFIX_LEN

NB = 64
N_WORKERS = 32
B_PER_W = B // N_WORKERS


def _sc_cls_gather(label, cls_ctx):
    mesh = plsc.VectorSubcoreMesh(core_axis_name="c", subcore_axis_name="s")

    @functools.partial(
        pl.kernel,
        out_type=jax.ShapeDtypeStruct((B, N_CLS, CTX_DIM), jnp.float32),
        mesh=mesh,
        scratch_types=[
            pltpu.VMEM((B_PER_W,), jnp.int32),
            pltpu.VMEM((B_PER_W, N_CLS, CTX_DIM), jnp.float32),
            pltpu.SemaphoreType.DMA,
        ],
    )
    def gather_kernel(label_hbm, cls_hbm, out_hbm, idx_v, rows_v, sem):
        wid = lax.axis_index("s") * 2 + lax.axis_index("c")
        base = wid * B_PER_W
        pltpu.sync_copy(label_hbm.at[pl.ds(base, B_PER_W)], idx_v)
        pltpu.async_copy(cls_hbm.at[idx_v], rows_v, sem).wait()
        pltpu.sync_copy(rows_v, out_hbm.at[pl.ds(base, B_PER_W)])

    return gather_kernel(label, cls_ctx)


NRING = 5
GRID = B // NB


def _tc_assemble_body(cls_ref, pre_ref, suf_ref, hi_ref, lo_ref, sel_ref,
                      out_ref, stage, sem):
    i = pl.program_id(0)
    slot = lax.rem(i, NRING)

    @pl.when(i >= NRING)
    def _():
        pltpu.make_async_copy(stage.at[slot], out_ref.at[pl.ds(0, NB)],
                              sem.at[slot]).wait()

    st = stage.at[slot]
    st[:, 0:PREFIX_LEN, :] = jnp.broadcast_to(
        pre_ref[...][None], (NB, PREFIX_LEN, CTX_DIM))
    st[:, PREFIX_LEN:PREFIX_LEN + N_CLS, :] = cls_ref[...]
    st[:, SEQ - SUFFIX_LEN:SEQ, :] = jnp.broadcast_to(
        suf_ref[...][None], (NB, SUFFIX_LEN, CTX_DIM))
    sel = sel_ref[...]
    onehot = (sel == lax.broadcasted_iota(jnp.int32, (1, SMALL_ROWS), 1))
    onehot = onehot.astype(jnp.bfloat16)
    res = (jnp.dot(onehot, hi_ref[...], preferred_element_type=jnp.float32)
           + jnp.dot(onehot, lo_ref[...], preferred_element_type=jnp.float32))
    a0 = PREFIX_LEN + N_CLS
    for b in range(NB):
        st[b, a0:a0 + N_ATTR_SLOTS, :] = (
            res[b * N_ATTR_SLOTS:(b + 1) * N_ATTR_SLOTS, :])

    pltpu.make_async_copy(stage.at[slot], out_ref.at[pl.ds(i * NB, NB)],
                          sem.at[slot]).start()

    @pl.when(i == GRID - 1)
    def _():
        for r in range(NRING):
            pltpu.make_async_copy(stage.at[r], out_ref.at[pl.ds(0, NB)],
                                  sem.at[r]).wait()


def _tc_assemble(cls_part, prefix, suffix, small_hi, small_lo, attr_sel):
    return pl.pallas_call(
        _tc_assemble_body,
        grid=(GRID,),
        in_specs=[
            pl.BlockSpec((NB, N_CLS, CTX_DIM), lambda i: (i, 0, 0)),
            pl.BlockSpec((PREFIX_LEN, CTX_DIM), lambda i: (0, 0)),
            pl.BlockSpec((SUFFIX_LEN, CTX_DIM), lambda i: (0, 0)),
            pl.BlockSpec((SMALL_ROWS, CTX_DIM), lambda i: (0, 0)),
            pl.BlockSpec((SMALL_ROWS, CTX_DIM), lambda i: (0, 0)),
            pl.BlockSpec((NB * N_ATTR_SLOTS, 1), lambda i: (i, 0)),
        ],
        out_specs=pl.BlockSpec(memory_space=pl.ANY),
        out_shape=jax.ShapeDtypeStruct((B, SEQ, CTX_DIM), jnp.float32),
        scratch_shapes=[
            pltpu.VMEM((NRING, NB, SEQ, CTX_DIM), jnp.float32),
            pltpu.SemaphoreType.DMA((NRING,)),
        ],
    )(cls_part, prefix, suffix, small_hi, small_lo, attr_sel)


def kernel(get_train, label, gender_idx, ucc_idx, ucs_idx, lcc_idx, lcs_idx,
           hat_idx, backpack_idx, token_prefix, token_suffix, gender_ctx,
           ucc_ctx, ucs_ctx, lcc_ctx, lcs_ctx, hat_ctx, backpack_ctx,
           cls_ctx):
    del get_train
    tables = (gender_ctx, ucc_ctx, ucs_ctx, lcc_ctx, lcs_ctx, hat_ctx,
              backpack_ctx)
    idxs = (gender_idx, ucc_idx, ucs_idx, lcc_idx, lcs_idx, hat_idx,
            backpack_idx)
    small = jnp.concatenate([t.reshape(-1, CTX_DIM) for t in tables], axis=0)
    small_hi = small.astype(jnp.bfloat16)
    small_lo = (small - small_hi.astype(jnp.float32)).astype(jnp.bfloat16)

    sel_parts = []
    base = 0
    j = jnp.arange(4, dtype=jnp.int32)[None, :]
    for t, ix in zip(tables, idxs):
        sel_parts.append(base + 4 * ix.astype(jnp.int32)[:, None] + j)
        base += t.shape[0] * 4
    attr_sel = jnp.concatenate(sel_parts, axis=1).reshape(
        B * N_ATTR_SLOTS, 1)

    prefix = token_prefix.reshape(PREFIX_LEN, CTX_DIM)
    suffix = token_suffix.reshape(SUFFIX_LEN, CTX_DIM)

    cls_part = _sc_cls_gather(label.astype(jnp.int32), cls_ctx)

    return _tc_assemble(cls_part, prefix, suffix, small_hi, small_lo,
                        attr_sel)
